# baseline (speedup 1.0000x reference)
import functools
import jax
import jax.numpy as jnp
from jax import lax
from jax.experimental import pallas as pl
from jax.experimental.pallas import tpu as pltpu
from jax.experimental.pallas import tpu_sc as plsc

_GLOBAL_WEIGHT = 1.0
_LOCAL_WEIGHT = 2.0

_N = 2 * 128 * 128 * 128
_NW = 32
_PER_W = _N // _NW
_CHUNK = 8192
_NCH = _PER_W // _CHUNK
_GROUPS = _CHUNK // 64
_CW = _CHUNK // 4


def _sc_body(p_hbm, o_hbm, g_hbm, l_hbm, out_hbm,
             p_buf, o_buf, g_buf, l_buf, stage,
             p_sem, o_sem, g_sem, l_sem):
    wid = lax.axis_index("s") * 2 + lax.axis_index("c")
    ebase = wid * _PER_W
    wbase = ebase // 4

    iota = lax.iota(jnp.int32, 16)
    qtr = lax.shift_right_logical(iota, 2)
    shf = (iota & 3) * 8
    one = jnp.int32(1)
    ff = jnp.int32(0xFF)

    def copies(c, s):
        eo = pl.multiple_of(ebase + c * _CHUNK, 512)
        wo = pl.multiple_of(wbase + c * _CW, 128)
        so = pl.multiple_of(s * _CHUNK, 512)
        sw = pl.multiple_of(s * _CW, 128)
        return (
            pltpu.make_async_copy(
                p_hbm.at[pl.ds(eo, _CHUNK)],
                p_buf.at[pl.ds(so, _CHUNK)], p_sem),
            pltpu.make_async_copy(
                o_hbm.at[pl.ds(eo, _CHUNK)],
                o_buf.at[pl.ds(so, _CHUNK)], o_sem),
            pltpu.make_async_copy(
                g_hbm.at[pl.ds(wo, _CW)],
                g_buf.at[pl.ds(sw, _CW)], g_sem),
            pltpu.make_async_copy(
                l_hbm.at[pl.ds(wo, _CW)],
                l_buf.at[pl.ds(sw, _CW)], l_sem),
        )

    def start(c, s):
        for cp in copies(c, s):
            cp.start()

    start(0, 0)
    start(1, 1)

    def chunk_body(c, accs):
        acc_g, acc_l, acc_cg, acc_cl = accs
        s = c % 2
        for cp in copies(c, s):
            cp.wait()

        eoff = s * _CHUNK
        woff = s * _CW

        zi = jnp.zeros((16,), jnp.int32)

        @plsc.parallel_loop(0, _GROUPS, carry=(acc_g, acc_l, zi, zi), unroll=8)
        def _groups(t, carry):
            a_g, a_l, cw_g, cw_l = carry
            wb = woff + t * 16
            wg = g_buf[pl.ds(wb, 16)]
            wl = l_buf[pl.ds(wb, 16)]
            gow = wg & (~wl)
            cw_g = cw_g + gow
            cw_l = cw_l + wl
            for k in range(4):
                idxk = 4 * k + qtr
                gok = gow.at[idxk].get(mode="promise_in_bounds")
                wlk = wl.at[idxk].get(mode="promise_in_bounds")
                gb = ((lax.shift_right_logical(gok, shf)) & one).astype(jnp.float32)
                lb = ((lax.shift_right_logical(wlk, shf)) & one).astype(jnp.float32)
                eb = eoff + t * 64 + 16 * k
                pv = p_buf[pl.ds(eb, 16)]
                ov = o_buf[pl.ds(eb, 16)]
                d = pv - ov
                d2 = d * d
                a_g = a_g + d2 * gb
                a_l = a_l + d2 * lb
            return (a_g, a_l, cw_g, cw_l)

        a_g, a_l, cw_g, cw_l = _groups

        def bytesum(w):
            return ((w & ff)
                    + (lax.shift_right_logical(w, 8) & ff)
                    + (lax.shift_right_logical(w, 16) & ff)
                    + (lax.shift_right_logical(w, 24) & ff))

        acc_cg = acc_cg + bytesum(cw_g).astype(jnp.float32)
        acc_cl = acc_cl + bytesum(cw_l).astype(jnp.float32)

        @pl.when(c + 2 < _NCH)
        def _next():
            start(c + 2, s)

        return (a_g, a_l, acc_cg, acc_cl)

    zf = jnp.zeros((16,), jnp.float32)
    acc_g, acc_l, acc_cg, acc_cl = lax.fori_loop(
        0, _NCH, chunk_body, (zf, zf, zf, zf))

    stage[0, :] = acc_g
    stage[1, :] = acc_cg
    stage[2, :] = acc_l
    stage[3, :] = acc_cl
    pltpu.sync_copy(stage, out_hbm.at[wid])


def _make_sc_call():
    mesh = plsc.VectorSubcoreMesh(core_axis_name="c", subcore_axis_name="s")
    return functools.partial(
        pl.kernel,
        mesh=mesh,
        out_type=jax.ShapeDtypeStruct((_NW, 4, 16), jnp.float32),
        scratch_types=[
            pltpu.VMEM((2 * _CHUNK,), jnp.float32),
            pltpu.VMEM((2 * _CHUNK,), jnp.float32),
            pltpu.VMEM((2 * _CW,), jnp.int32),
            pltpu.VMEM((2 * _CW,), jnp.int32),
            pltpu.VMEM((4, 16), jnp.float32),
            pltpu.SemaphoreType.DMA,
            pltpu.SemaphoreType.DMA,
            pltpu.SemaphoreType.DMA,
            pltpu.SemaphoreType.DMA,
        ],
    )(_sc_body)


_sc_call = _make_sc_call()


def kernel(predicted_image, original_image, global_mask, local_mask):
    p = predicted_image.reshape(-1)
    o = original_image.reshape(-1)
    gm = global_mask.reshape(-1).view(jnp.int32)
    lm = local_mask.reshape(-1).view(jnp.int32)

    parts = _sc_call(p, o, gm, lm)
    sums = parts.sum(axis=(0, 2))

    global_loss = sums[0] / (sums[1] + 1e-08)
    local_loss = sums[2] / (sums[3] + 1e-08)
    total_loss = _GLOBAL_WEIGHT * global_loss + _LOCAL_WEIGHT * local_loss
    return (total_loss, global_loss, local_loss)


# speedup vs baseline: 12.5149x; 12.5149x over previous
import functools
import jax
import jax.numpy as jnp
from jax import lax
from jax.experimental import pallas as pl
from jax.experimental.pallas import tpu as pltpu
from jax.experimental.pallas import tpu_sc as plsc

_GLOBAL_WEIGHT = 1.0
_LOCAL_WEIGHT = 2.0

_COLS = 128
_NBLK = 512
_NW = 32
_WBLK = _NBLK // _NW
_R4 = 16
_RJ = 4


def _sc_body(p_hbm, o_hbm, g_hbm, l_hbm, out_hbm,
             p_buf, o_buf, g_buf, l_buf, stage,
             p_sem, o_sem, g_sem, l_sem):
    wid = lax.axis_index("s") * 2 + lax.axis_index("c")
    bbase = wid * _WBLK

    iota = lax.iota(jnp.int32, 16)
    qtr = lax.shift_right_logical(iota, 2)
    shf = (iota & 3) * 8
    one = jnp.int32(1)
    ff = jnp.int32(0xFF)

    g_hbm32 = g_hbm.bitcast(jnp.int32)
    l_hbm32 = l_hbm.bitcast(jnp.int32)

    def copies(c, s):
        b = bbase + c
        return (
            pltpu.make_async_copy(p_hbm.at[b], p_buf.at[s], p_sem),
            pltpu.make_async_copy(o_hbm.at[b], o_buf.at[s], o_sem),
            pltpu.make_async_copy(g_hbm32.at[b], g_buf.at[s], g_sem),
            pltpu.make_async_copy(l_hbm32.at[b], l_buf.at[s], l_sem),
        )

    def start(c, s):
        for cp in copies(c, s):
            cp.start()

    start(0, 0)
    start(1, 1)

    def chunk_body(c, accs):
        acc_g, acc_l, acc_cg, acc_cl = accs
        s = c % 2
        for cp in copies(c, s):
            cp.wait()

        zi = jnp.zeros((16,), jnp.int32)

        @plsc.parallel_loop(0, _R4, carry=(acc_g, acc_l, zi, zi), unroll=2)
        def _rows(r4, carry):
            a_g, a_l, cw_g, cw_l = carry
            for j in range(_RJ):
                wg0 = g_buf[s, r4, 0, pl.ds(32 * j, 16)]
                wg1 = g_buf[s, r4, 0, pl.ds(32 * j + 16, 16)]
                wl0 = l_buf[s, r4, 0, pl.ds(32 * j, 16)]
                wl1 = l_buf[s, r4, 0, pl.ds(32 * j + 16, 16)]
                go0 = wg0 & (~wl0)
                go1 = wg1 & (~wl1)
                cw_g = cw_g + go0 + go1
                cw_l = cw_l + wl0 + wl1
                for k in range(8):
                    go = go0 if k < 4 else go1
                    wl = wl0 if k < 4 else wl1
                    idxk = 4 * (k % 4) + qtr
                    gok = go.at[idxk].get(mode="promise_in_bounds")
                    wlk = wl.at[idxk].get(mode="promise_in_bounds")
                    gb = ((lax.shift_right_logical(gok, shf)) & one).astype(jnp.float32)
                    lb = ((lax.shift_right_logical(wlk, shf)) & one).astype(jnp.float32)
                    pv = p_buf[s, r4, j, pl.ds(16 * k, 16)]
                    ov = o_buf[s, r4, j, pl.ds(16 * k, 16)]
                    d = pv - ov
                    d2 = d * d
                    a_g = a_g + d2 * gb
                    a_l = a_l + d2 * lb
            return (a_g, a_l, cw_g, cw_l)

        a_g, a_l, cw_g, cw_l = _rows

        def bytesum(w):
            return ((w & ff)
                    + (lax.shift_right_logical(w, 8) & ff)
                    + (lax.shift_right_logical(w, 16) & ff)
                    + (lax.shift_right_logical(w, 24) & ff))

        acc_cg = acc_cg + bytesum(cw_g).astype(jnp.float32)
        acc_cl = acc_cl + bytesum(cw_l).astype(jnp.float32)

        @pl.when(c + 2 < _WBLK)
        def _next():
            start(c + 2, s)

        return (a_g, a_l, acc_cg, acc_cl)

    zf = jnp.zeros((16,), jnp.float32)
    acc_g, acc_l, acc_cg, acc_cl = lax.fori_loop(
        0, _WBLK, chunk_body, (zf, zf, zf, zf))

    stage[0, :] = acc_g
    stage[1, :] = acc_cg
    stage[2, :] = acc_l
    stage[3, :] = acc_cl
    pltpu.sync_copy(stage, out_hbm.at[wid])


def _make_sc_call():
    mesh = plsc.VectorSubcoreMesh(core_axis_name="c", subcore_axis_name="s")
    return functools.partial(
        pl.kernel,
        mesh=mesh,
        out_type=jax.ShapeDtypeStruct((_NW, 4, 16), jnp.float32),
        scratch_types=[
            pltpu.VMEM((2, _R4, _RJ, _COLS), jnp.float32),
            pltpu.VMEM((2, _R4, _RJ, _COLS), jnp.float32),
            pltpu.VMEM((2, _R4, 1, _COLS), jnp.int32),
            pltpu.VMEM((2, _R4, 1, _COLS), jnp.int32),
            pltpu.VMEM((4, 16), jnp.float32),
            pltpu.SemaphoreType.DMA,
            pltpu.SemaphoreType.DMA,
            pltpu.SemaphoreType.DMA,
            pltpu.SemaphoreType.DMA,
        ],
    )(_sc_body)


_sc_call = _make_sc_call()


def kernel(predicted_image, original_image, global_mask, local_mask):
    p = predicted_image.reshape(_NBLK, _R4, _RJ, _COLS)
    o = original_image.reshape(_NBLK, _R4, _RJ, _COLS)
    gm = global_mask.view(jnp.int8).reshape(_NBLK, _R4, _RJ, _COLS)
    lm = local_mask.view(jnp.int8).reshape(_NBLK, _R4, _RJ, _COLS)

    parts = _sc_call(p, o, gm, lm)
    sums = parts.sum(axis=(0, 2))

    global_loss = sums[0] / (sums[1] + 1e-08)
    local_loss = sums[2] / (sums[3] + 1e-08)
    total_loss = _GLOBAL_WEIGHT * global_loss + _LOCAL_WEIGHT * local_loss
    return (total_loss, global_loss, local_loss)


# speedup vs baseline: 15.1661x; 1.2118x over previous
import functools
import jax
import jax.numpy as jnp
from jax import lax
from jax.experimental import pallas as pl
from jax.experimental.pallas import tpu as pltpu
from jax.experimental.pallas import tpu_sc as plsc

_GLOBAL_WEIGHT = 1.0
_LOCAL_WEIGHT = 2.0

_COLS = 128
_NBLK = 512
_NW = 32
_WBLK = _NBLK // _NW
_R4 = 16
_RJ = 4


def _sc_body(p_hbm, o_hbm, g_hbm, l_hbm, out_hbm,
             p_buf, o_buf, g_buf, l_buf, stage,
             p_sem, o_sem, g_sem, l_sem):
    wid = lax.axis_index("s") * 2 + lax.axis_index("c")
    bbase = wid * _WBLK

    one = jnp.int32(1)
    ff = jnp.int32(0xFF)

    g_hbm32 = g_hbm.bitcast(jnp.int32)
    l_hbm32 = l_hbm.bitcast(jnp.int32)

    def copies(c, s):
        b = bbase + c
        return (
            pltpu.make_async_copy(p_hbm.at[b], p_buf.at[s], p_sem),
            pltpu.make_async_copy(o_hbm.at[b], o_buf.at[s], o_sem),
            pltpu.make_async_copy(g_hbm32.at[b], g_buf.at[s], g_sem),
            pltpu.make_async_copy(l_hbm32.at[b], l_buf.at[s], l_sem),
        )

    def start(c, s):
        for cp in copies(c, s):
            cp.start()

    start(0, 0)
    start(1, 1)

    def chunk_body(c, accs):
        acc_g, acc_l, acc_cg, acc_cl = accs
        s = c % 2
        for cp in copies(c, s):
            cp.wait()

        zi = jnp.zeros((16,), jnp.int32)

        @plsc.parallel_loop(0, _R4, carry=(acc_g, acc_l, zi, zi), unroll=2)
        def _rows(r4, carry):
            a_g, a_l, cw_g, cw_l = carry
            for k in range(8):
                wgk = g_buf[s, r4, 0, pl.ds(16 * k, 16)]
                wlk = l_buf[s, r4, 0, pl.ds(16 * k, 16)]
                gok = wgk & (~wlk)
                cw_g = cw_g + gok
                cw_l = cw_l + wlk
                for j in range(_RJ):
                    gb = ((gok >> (8 * j)) & one).astype(jnp.float32)
                    lb = ((wlk >> (8 * j)) & one).astype(jnp.float32)
                    pv = p_buf[s, r4, j, pl.ds(16 * k, 16)]
                    ov = o_buf[s, r4, j, pl.ds(16 * k, 16)]
                    d = pv - ov
                    d2 = d * d
                    a_g = a_g + d2 * gb
                    a_l = a_l + d2 * lb
            return (a_g, a_l, cw_g, cw_l)

        a_g, a_l, cw_g, cw_l = _rows

        def bytesum(w):
            return ((w & ff)
                    + (lax.shift_right_logical(w, 8) & ff)
                    + (lax.shift_right_logical(w, 16) & ff)
                    + (lax.shift_right_logical(w, 24) & ff))

        acc_cg = acc_cg + bytesum(cw_g).astype(jnp.float32)
        acc_cl = acc_cl + bytesum(cw_l).astype(jnp.float32)

        @pl.when(c + 2 < _WBLK)
        def _next():
            start(c + 2, s)

        return (a_g, a_l, acc_cg, acc_cl)

    zf = jnp.zeros((16,), jnp.float32)
    acc_g, acc_l, acc_cg, acc_cl = lax.fori_loop(
        0, _WBLK, chunk_body, (zf, zf, zf, zf))

    stage[0, :] = acc_g
    stage[1, :] = acc_cg
    stage[2, :] = acc_l
    stage[3, :] = acc_cl
    pltpu.sync_copy(stage, out_hbm.at[wid])


def _make_sc_call():
    mesh = plsc.VectorSubcoreMesh(core_axis_name="c", subcore_axis_name="s")
    return functools.partial(
        pl.kernel,
        mesh=mesh,
        out_type=jax.ShapeDtypeStruct((_NW, 4, 16), jnp.float32),
        scratch_types=[
            pltpu.VMEM((2, _R4, _RJ, _COLS), jnp.float32),
            pltpu.VMEM((2, _R4, _RJ, _COLS), jnp.float32),
            pltpu.VMEM((2, _R4, 1, _COLS), jnp.int32),
            pltpu.VMEM((2, _R4, 1, _COLS), jnp.int32),
            pltpu.VMEM((4, 16), jnp.float32),
            pltpu.SemaphoreType.DMA,
            pltpu.SemaphoreType.DMA,
            pltpu.SemaphoreType.DMA,
            pltpu.SemaphoreType.DMA,
        ],
    )(_sc_body)


_sc_call = _make_sc_call()


def kernel(predicted_image, original_image, global_mask, local_mask):
    p = predicted_image.reshape(_NBLK, _R4, _RJ, _COLS)
    o = original_image.reshape(_NBLK, _R4, _RJ, _COLS)
    gm = global_mask.view(jnp.int8).reshape(_NBLK, _R4, _RJ, _COLS)
    lm = local_mask.view(jnp.int8).reshape(_NBLK, _R4, _RJ, _COLS)

    parts = _sc_call(p, o, gm, lm)
    sums = parts.sum(axis=(0, 2))

    global_loss = sums[0] / (sums[1] + 1e-08)
    local_loss = sums[2] / (sums[3] + 1e-08)
    total_loss = _GLOBAL_WEIGHT * global_loss + _LOCAL_WEIGHT * local_loss
    return (total_loss, global_loss, local_loss)
